# baseline (speedup 1.0000x reference)
import functools

import jax
import jax.numpy as jnp
from jax import lax
from jax.experimental import pallas as pl
from jax.experimental.pallas import tpu as pltpu
from jax.experimental.pallas import tpu_sc as plsc

NC, NS = 2, 16
NW = NC * NS
CH = 512
Q = 1
NB = 2


def _cdiv(a, b):
    return -(-a // b)


@functools.lru_cache(maxsize=None)
def _make_seg_sum(width, groups, npad, with_counts):
    assert groups % NB == 0
    zr = npad // NS
    outs = [jax.ShapeDtypeStruct((NC, npad, width), jnp.float32)]
    scratch = [
        pltpu.VMEM((NB, Q, CH), jnp.int32),
        pltpu.VMEM((NB, Q, CH), jnp.int32),
        pltpu.VMEM((NB, Q, CH, width), jnp.float32),
        pltpu.VMEM_SHARED((npad, width), jnp.float32),
    ] + [pltpu.SemaphoreType.DMA] * (2 * NB)
    if with_counts:
        outs.append(jax.ShapeDtypeStruct((NC, npad, 16), jnp.float32))
        scratch += [
            pltpu.VMEM((CH, 16), jnp.float32),
            pltpu.VMEM_SHARED((npad, 16), jnp.float32),
        ]
    mesh = plsc.VectorSubcoreMesh(core_axis_name="c", subcore_axis_name="s")

    def body(*refs):
        if with_counts:
            (table, srcp, dstp, zrow, zcnt, ones, part, cpart,
             sidx, didx, rows, acc, *rest) = refs
            gsem, ssem = rest[:NB], rest[NB:2 * NB]
            onesv, cacc = rest[2 * NB], rest[2 * NB + 1]
        else:
            (table, srcp, dstp, zrow, part,
             sidx, didx, rows, acc, *rest) = refs
            gsem, ssem = rest[:NB], rest[NB:2 * NB]
        c = lax.axis_index("c")
        s = lax.axis_index("s")
        w = s * NC + c

        pltpu.sync_copy(zrow, acc.at[pl.ds(s * zr, zr)])
        if with_counts:
            pltpu.sync_copy(zcnt, cacc.at[pl.ds(s * zr, zr)])
            pltpu.sync_copy(ones, onesv)
        plsc.subcore_barrier()

        def iteration(i, carry):
            g0 = i * NB
            pltpu.sync_copy(srcp.at[w, pl.ds(g0, NB)], sidx)
            pltpu.sync_copy(dstp.at[w, pl.ds(g0, NB)], didx)
            gds = [[pltpu.async_copy(table.at[sidx.at[k, q]],
                                     rows.at[k, q], gsem[k])
                    for q in range(Q)] for k in range(NB)]
            sds = []
            for k in range(NB):
                for d in gds[k]:
                    d.wait()
                for q in range(Q):
                    sds.append(pltpu.async_copy(
                        rows.at[k, q], acc.at[didx.at[k, q]], ssem[k],
                        add=True))
                    if with_counts:
                        sds.append(pltpu.async_copy(
                            onesv, cacc.at[didx.at[k, q]], ssem[k], add=True))
            for d in sds:
                d.wait()
            return carry

        lax.fori_loop(0, groups // NB, iteration, 0)
        plsc.subcore_barrier()

        pltpu.sync_copy(acc.at[pl.ds(s * zr, zr)],
                        part.at[c, pl.ds(s * zr, zr)])
        if with_counts:
            pltpu.sync_copy(cacc.at[pl.ds(s * zr, zr)],
                            cpart.at[c, pl.ds(s * zr, zr)])

    return pl.kernel(
        body, out_type=outs, mesh=mesh, scratch_types=scratch,
        compiler_params=pltpu.CompilerParams(use_tc_tiling_on_sc=False))


def _mm2_body(x_ref, wl_ref, wr_ref, yl_ref, yr_ref):
    xv = x_ref[...]
    yl_ref[...] = jnp.dot(xv, wl_ref[...], preferred_element_type=jnp.float32)
    yr_ref[...] = jnp.dot(xv, wr_ref[...], preferred_element_type=jnp.float32)


def _mid_body(n, part_ref, cpart_ref, b_ref, xr_ref, wl_ref, wr_ref,
              y2_ref, hr_ref, cnt_ref):
    p = part_ref[0, :n, :] + part_ref[1, :n, :]
    cnt = jnp.maximum(cpart_ref[0, :n, 0:1] + cpart_ref[1, :n, 0:1], 1.0)
    hh = jnp.maximum(p / cnt + b_ref[...] + xr_ref[...], 0.0)
    y2_ref[...] = jnp.dot(hh, wl_ref[...], preferred_element_type=jnp.float32)
    hr_ref[...] = jnp.dot(hh, wr_ref[...], preferred_element_type=jnp.float32)
    cnt_ref[...] = jnp.broadcast_to(cnt, (n, 16))


def _fin_body(n, part_ref, cnt_ref, b_ref, hr_ref, wf_ref, bf_ref, out_ref):
    p = part_ref[0, :n, :] + part_ref[1, :n, :]
    h = jnp.maximum(p / cnt_ref[:, 0:1] + b_ref[...] + hr_ref[...], 0.0)
    out_ref[...] = (jnp.dot(h, wf_ref[...], preferred_element_type=jnp.float32)
                    + bf_ref[...])


def kernel(x, edge_index, W1l, b1, W1r, W2l, b2, W2r, Wf, bf):
    n, d = x.shape
    h = W1l.shape[1]
    e = edge_index.shape[1]

    grp = NW * Q * CH
    groups = _cdiv(_cdiv(e, grp), NB) * NB
    epad = groups * grp
    npad = (_cdiv(n + 1, NS * 8) + 1) * (NS * 8)
    zr = npad // NS

    src = edge_index[0].astype(jnp.int32)
    dst = edge_index[1].astype(jnp.int32)
    pad = epad - e
    padd = n + (jnp.arange(pad, dtype=jnp.int32) % (npad - n))
    srcp = (jnp.concatenate([src, jnp.zeros((pad,), jnp.int32)])
            .reshape(-1, NW).T.reshape(NW, groups, Q, CH))
    dstp = (jnp.concatenate([dst, padd])
            .reshape(-1, NW).T.reshape(NW, groups, Q, CH))
    z64 = jnp.zeros((zr, h), jnp.float32)
    zcnt = jnp.zeros((zr, 16), jnp.float32)
    ones = jnp.ones((CH, 16), jnp.float32)

    f32 = jnp.float32

    y1, xr = pl.pallas_call(
        _mm2_body,
        out_shape=[jax.ShapeDtypeStruct((n, h), f32),
                   jax.ShapeDtypeStruct((n, h), f32)],
    )(x, W1l, W1r)

    part1, cpart = _make_seg_sum(h, groups, npad, True)(
        y1, srcp, dstp, z64, zcnt, ones)

    y2, hr, cnt = pl.pallas_call(
        functools.partial(_mid_body, n),
        out_shape=[jax.ShapeDtypeStruct((n, h), f32),
                   jax.ShapeDtypeStruct((n, h), f32),
                   jax.ShapeDtypeStruct((n, 16), f32)],
    )(part1, cpart, b1.reshape(1, h), xr, W2l, W2r)

    (part2,) = _make_seg_sum(h, groups, npad, False)(y2, srcp, dstp, z64)

    wfp = jnp.pad(Wf, ((0, 0), (0, 128 - Wf.shape[1])))
    bfp = jnp.broadcast_to(bf.reshape(1, 1), (1, 128))
    out_pad = pl.pallas_call(
        functools.partial(_fin_body, n),
        out_shape=jax.ShapeDtypeStruct((n, 128), f32),
    )(part2, cnt, b2.reshape(1, h), hr, wfp, bfp)

    return out_pad[:, :Wf.shape[1]]


# speedup vs baseline: 1.0173x; 1.0173x over previous
import functools

import jax
import jax.numpy as jnp
from jax import lax
from jax.experimental import pallas as pl
from jax.experimental.pallas import tpu as pltpu
from jax.experimental.pallas import tpu_sc as plsc

NC, NS = 2, 16
NW = NC * NS
CH = 128
Q = 4
NB = 2


def _cdiv(a, b):
    return -(-a // b)


@functools.lru_cache(maxsize=None)
def _make_seg_sum(width, groups, npad, with_counts):
    assert groups % NB == 0
    zr = npad // NS
    outs = [jax.ShapeDtypeStruct((NC, npad, width), jnp.float32)]
    scratch = [
        pltpu.VMEM((NB, Q, CH), jnp.int32),
        pltpu.VMEM((NB, Q, CH), jnp.int32),
        pltpu.VMEM((NB, Q, CH, width), jnp.float32),
        pltpu.VMEM_SHARED((npad, width), jnp.float32),
    ] + [pltpu.SemaphoreType.DMA] * (3 * NB)
    if with_counts:
        outs.append(jax.ShapeDtypeStruct((NC, npad, 16), jnp.float32))
        scratch += [
            pltpu.VMEM((CH, 16), jnp.float32),
            pltpu.VMEM_SHARED((npad, 16), jnp.float32),
        ]
    mesh = plsc.VectorSubcoreMesh(core_axis_name="c", subcore_axis_name="s")

    def body(*refs):
        if with_counts:
            (table, srcp, dstp, zrow, zcnt, ones, part, cpart,
             sidx, didx, rows, acc, *rest) = refs
            gsem, ssem, isem = (rest[:NB], rest[NB:2 * NB],
                                rest[2 * NB:3 * NB])
            onesv, cacc = rest[3 * NB], rest[3 * NB + 1]
        else:
            (table, srcp, dstp, zrow, part,
             sidx, didx, rows, acc, *rest) = refs
            gsem, ssem, isem = (rest[:NB], rest[NB:2 * NB],
                                rest[2 * NB:3 * NB])
        c = lax.axis_index("c")
        s = lax.axis_index("s")
        w = s * NC + c

        pltpu.sync_copy(zrow, acc.at[pl.ds(s * zr, zr)])
        if with_counts:
            pltpu.sync_copy(zcnt, cacc.at[pl.ds(s * zr, zr)])
            pltpu.sync_copy(ones, onesv)
        plsc.subcore_barrier()

        def fire_scatters(k):
            descs = []
            for q in range(Q):
                descs.append(pltpu.async_copy(
                    rows.at[k, q], acc.at[didx.at[k, q]], ssem[k], add=True))
                if with_counts:
                    descs.append(pltpu.async_copy(
                        onesv, cacc.at[didx.at[k, q]], ssem[k], add=True))
            return descs

        def iteration(i, carry):
            g0 = i * NB
            pltpu.sync_copy(srcp.at[w, g0], sidx.at[0])
            pltpu.sync_copy(dstp.at[w, g0], didx.at[0])
            gds = [pltpu.async_copy(table.at[sidx.at[0, q]], rows.at[0, q],
                                    gsem[0]) for q in range(Q)]
            idxds = [
                (pltpu.async_copy(srcp.at[w, g0 + k], sidx.at[k], isem[k]),
                 pltpu.async_copy(dstp.at[w, g0 + k], didx.at[k], isem[k]))
                for k in range(1, NB)
            ]
            sds = []
            for k in range(NB):
                for d in gds:
                    d.wait()
                sds += fire_scatters(k)
                if k + 1 < NB:
                    for d in idxds[k]:
                        d.wait()
                    gds = [pltpu.async_copy(table.at[sidx.at[k + 1, q]],
                                            rows.at[k + 1, q], gsem[k + 1])
                           for q in range(Q)]
            for d in sds:
                d.wait()
            return carry

        lax.fori_loop(0, groups // NB, iteration, 0)
        plsc.subcore_barrier()

        pltpu.sync_copy(acc.at[pl.ds(s * zr, zr)],
                        part.at[c, pl.ds(s * zr, zr)])
        if with_counts:
            pltpu.sync_copy(cacc.at[pl.ds(s * zr, zr)],
                            cpart.at[c, pl.ds(s * zr, zr)])

    return pl.kernel(
        body, out_type=outs, mesh=mesh, scratch_types=scratch,
        compiler_params=pltpu.CompilerParams(use_tc_tiling_on_sc=False))


def _mm2_body(x_ref, wl_ref, wr_ref, yl_ref, yr_ref):
    xv = x_ref[...]
    yl_ref[...] = jnp.dot(xv, wl_ref[...], preferred_element_type=jnp.float32)
    yr_ref[...] = jnp.dot(xv, wr_ref[...], preferred_element_type=jnp.float32)


def _mid_body(n, part_ref, cpart_ref, b_ref, xr_ref, wl_ref, wr_ref,
              y2_ref, hr_ref, cnt_ref):
    p = part_ref[0, :n, :] + part_ref[1, :n, :]
    cnt = jnp.maximum(cpart_ref[0, :n, 0:1] + cpart_ref[1, :n, 0:1], 1.0)
    hh = jnp.maximum(p / cnt + b_ref[...] + xr_ref[...], 0.0)
    y2_ref[...] = jnp.dot(hh, wl_ref[...], preferred_element_type=jnp.float32)
    hr_ref[...] = jnp.dot(hh, wr_ref[...], preferred_element_type=jnp.float32)
    cnt_ref[...] = jnp.broadcast_to(cnt, (n, 16))


def _fin_body(n, part_ref, cnt_ref, b_ref, hr_ref, wf_ref, bf_ref, out_ref):
    p = part_ref[0, :n, :] + part_ref[1, :n, :]
    h = jnp.maximum(p / cnt_ref[:, 0:1] + b_ref[...] + hr_ref[...], 0.0)
    out_ref[...] = (jnp.dot(h, wf_ref[...], preferred_element_type=jnp.float32)
                    + bf_ref[...])


def kernel(x, edge_index, W1l, b1, W1r, W2l, b2, W2r, Wf, bf):
    n, d = x.shape
    h = W1l.shape[1]
    e = edge_index.shape[1]

    grp = NW * Q * CH
    groups = _cdiv(_cdiv(e, grp), NB) * NB
    epad = groups * grp
    npad = (_cdiv(n + 1, NS * 8) + 1) * (NS * 8)
    zr = npad // NS

    src = edge_index[0].astype(jnp.int32)
    dst = edge_index[1].astype(jnp.int32)
    pad = epad - e
    padd = n + (jnp.arange(pad, dtype=jnp.int32) % (npad - n))
    srcp = (jnp.concatenate([src, jnp.zeros((pad,), jnp.int32)])
            .reshape(-1, NW).T.reshape(NW, groups, Q, CH))
    dstp = (jnp.concatenate([dst, padd])
            .reshape(-1, NW).T.reshape(NW, groups, Q, CH))
    z64 = jnp.zeros((zr, h), jnp.float32)
    zcnt = jnp.zeros((zr, 16), jnp.float32)
    ones = jnp.ones((CH, 16), jnp.float32)

    f32 = jnp.float32

    y1, xr = pl.pallas_call(
        _mm2_body,
        out_shape=[jax.ShapeDtypeStruct((n, h), f32),
                   jax.ShapeDtypeStruct((n, h), f32)],
    )(x, W1l, W1r)

    part1, cpart = _make_seg_sum(h, groups, npad, True)(
        y1, srcp, dstp, z64, zcnt, ones)

    y2, hr, cnt = pl.pallas_call(
        functools.partial(_mid_body, n),
        out_shape=[jax.ShapeDtypeStruct((n, h), f32),
                   jax.ShapeDtypeStruct((n, h), f32),
                   jax.ShapeDtypeStruct((n, 16), f32)],
    )(part1, cpart, b1.reshape(1, h), xr, W2l, W2r)

    (part2,) = _make_seg_sum(h, groups, npad, False)(y2, srcp, dstp, z64)

    wfp = jnp.pad(Wf, ((0, 0), (0, 128 - Wf.shape[1])))
    bfp = jnp.broadcast_to(bf.reshape(1, 1), (1, 128))
    out_pad = pl.pallas_call(
        functools.partial(_fin_body, n),
        out_shape=jax.ShapeDtypeStruct((n, 128), f32),
    )(part2, cnt, b2.reshape(1, h), hr, wfp, bfp)

    return out_pad[:, :Wf.shape[1]]
